# baseline (speedup 1.0000x reference)
import functools

import jax
import jax.numpy as jnp
from jax import lax
from jax.experimental import pallas as pl
from jax.experimental.pallas import tpu as pltpu
from jax.experimental.pallas import tpu_sc as plsc

_N = 10000
_E = 320000
_NP = 10240
_NW = 32
_BLK = 128
_S = 80
_EPAD = _NW * _S * _BLK
_RPT = _NP // 16
_RB = 1024
_GRID = 10

def _mesh():
    return plsc.VectorSubcoreMesh(core_axis_name="c", subcore_axis_name="s",
                                  num_cores=2, num_subcores=16)


def _leaky(v):
    return jnp.where(v > 0, v, 0.01 * v)


_W = 8


def _deg_body(ei_hbm, out_hbm, buf, cidx, acc, sems):
    c = lax.axis_index("c")
    t = lax.axis_index("s")
    w = c * 16 + t

    def fill(val):
        def body(r, _):
            buf[r, :] = jnp.full((16,), val, jnp.float32)
            return 0
        lax.fori_loop(0, _BLK, body, 0)

    fill(0.0)
    for k in range(_RPT // _BLK):
        pltpu.sync_copy(buf, acc.at[pl.ds(t * _RPT + k * _BLK, _BLK)])
    fill(1.0)
    pltpu.sync_copy(ei_hbm.at[1, pl.ds(w * _S, _S)], cidx)
    plsc.subcore_barrier()

    def scat(step, b):
        return pltpu.make_async_copy(buf, acc.at[cidx.at[step]], sems.at[b])

    for b in range(_W):
        scat(b, b).start(add=True)

    def group(i, _):
        for b in range(_W):
            scat(_W * i + b, b).wait()
            scat(_W * (i + 1) + b, b).start(add=True)
        return 0

    lax.fori_loop(0, _S // _W - 1, group, 0)
    for b in range(_W):
        scat(_S - _W + b, b).wait()
    plsc.subcore_barrier()
    pltpu.sync_copy(acc.at[pl.ds(t * _RPT, _RPT)],
                    out_hbm.at[c, pl.ds(t * _RPT, _RPT)])


def _deg_call(ei):
    fn = pl.kernel(
        _deg_body,
        out_type=jax.ShapeDtypeStruct((2, _NP, 16), jnp.float32),
        mesh=_mesh(),
        scratch_types=[
            pltpu.VMEM((_BLK, 16), jnp.float32),
            pltpu.VMEM((_S, _BLK), jnp.int32),
            pltpu.VMEM_SHARED((_NP, 16), jnp.float32),
            pltpu.SemaphoreType.DMA((_W,)),
        ],
        compiler_params=pltpu.CompilerParams(use_tc_tiling_on_sc=False),
    )
    return fn(ei)


def _agg_body(ei_hbm, y_hbm, out_hbm, ridx, cidx, *bufs_and_sems, d):
    gb = bufs_and_sems[:_W]
    acc = bufs_and_sems[_W]
    gsem = bufs_and_sems[_W + 1]
    ssem = bufs_and_sems[_W + 2]
    c = lax.axis_index("c")
    t = lax.axis_index("s")
    w = c * 16 + t

    def zrow(r, _):
        for j in range(d // 16):
            gb[0][r, pl.ds(j * 16, 16)] = jnp.zeros((16,), jnp.float32)
        return 0

    lax.fori_loop(0, _BLK, zrow, 0)
    for k in range(_RPT // _BLK):
        pltpu.sync_copy(gb[0], acc.at[pl.ds(t * _RPT + k * _BLK, _BLK)])
    pltpu.sync_copy(ei_hbm.at[0, pl.ds(w * _S, _S)], ridx)
    pltpu.sync_copy(ei_hbm.at[1, pl.ds(w * _S, _S)], cidx)
    plsc.subcore_barrier()

    def gath(step, b):
        return pltpu.make_async_copy(y_hbm.at[ridx.at[step]], gb[b],
                                     gsem.at[b])

    def scat(step, b):
        return pltpu.make_async_copy(gb[b], acc.at[cidx.at[step]],
                                     ssem.at[b])

    for b in range(_W):
        gath(b, b).start()

    def group(i, _):
        t0 = _W * i
        for b in range(_W):
            gath(t0 + b, b).wait()
            scat(t0 + b, b).start(add=True)
        for b in range(_W):
            scat(t0 + b, b).wait()
            gath(t0 + _W + b, b).start()
        return 0

    lax.fori_loop(0, _S // _W - 1, group, 0)
    t0 = _S - _W
    for b in range(_W):
        gath(t0 + b, b).wait()
        scat(t0 + b, b).start(add=True)
    for b in range(_W):
        scat(t0 + b, b).wait()
    plsc.subcore_barrier()
    pltpu.sync_copy(acc.at[pl.ds(t * _RPT, _RPT)],
                    out_hbm.at[c, pl.ds(t * _RPT, _RPT), pl.ds(0, d)])


def _agg_call(ei, y, d):
    fn = pl.kernel(
        functools.partial(_agg_body, d=d),
        out_type=jax.ShapeDtypeStruct((2, _NP, 128), jnp.float32),
        mesh=_mesh(),
        scratch_types=[
            pltpu.VMEM((_S, _BLK), jnp.int32),
            pltpu.VMEM((_S, _BLK), jnp.int32),
        ] + [pltpu.VMEM((_BLK, d), jnp.float32)] * _W + [
            pltpu.VMEM_SHARED((_NP, d), jnp.float32),
            pltpu.SemaphoreType.DMA((_W,)),
            pltpu.SemaphoreType.DMA((_W,)),
        ],
        compiler_params=pltpu.CompilerParams(use_tc_tiling_on_sc=False),
    )
    return fn(ei, y)


def _dis(dg0, dg1):
    return lax.rsqrt(dg0[0, :, 0:1] + dg1[0, :, 0:1] + 1.0)


def _pack(v, d):
    g = 128 // d
    r = v.reshape(v.shape[0] // g, g, d)
    return jnp.concatenate([r[:, k, :] for k in range(g)], axis=1)


def _unpack(p, d):
    g = 128 // d
    s = jnp.stack([p[:, k * d:(k + 1) * d] for k in range(g)], axis=1)
    return s.reshape(p.shape[0] * g, d)


def _tc1_body(x, w1, b1, w2, b2, wc1, dg0, dg1, y1):
    dis = _dis(dg0[:], dg1[:])
    h = _leaky(x[:] @ w1[:] + b1[:])
    h = _leaky(h @ w2[:] + b2[:])
    y1[:] = _pack((h @ wc1[:]) * dis, 64)


def _gru_block(g, p, wall, ball, h):
    m = jnp.concatenate([g, p], axis=1)
    gg = m @ wall + ball
    r = jax.nn.sigmoid(gg[:, :h])
    z = jax.nn.sigmoid(gg[:, h:2 * h])
    nn = jnp.tanh(gg[:, 2 * h:3 * h] + r * gg[:, 3 * h:])
    return (1.0 - z) * nn + z * p


def _tc2_body(a0, a1, y1, dg0, dg1, prev, bc, wall, ball, wc2, emb1, y2):
    dis = _dis(dg0[:], dg1[:])
    g = _leaky(dis * (a0[0][:, :64] + a1[0][:, :64] + _unpack(y1[:], 64))
               + bc[:])
    p = prev[:].T
    e = _gru_block(g, p, wall[:], ball[:], 64)
    emb1[:] = e.T
    y2[:] = _pack((e @ wc2[:]) * dis, 32)


def _tc3_body(a0, a1, y2, dg0, dg1, prev, bc, wall, ball, wp, bp, emb2, outv):
    dis = _dis(dg0[:], dg1[:])
    g = _leaky(dis * (a0[0][:, :32] + a1[0][:, :32] + _unpack(y2[:], 32))
               + bc[:])
    p = prev[:].T
    e = _gru_block(g, p, wall[:], ball[:], 32)
    emb2[:] = e.T
    outv[:] = jnp.sum(e @ wp[:] + bp[:], axis=1)


def _row_spec(d):
    return pl.BlockSpec((_RB, d), lambda i: (i, 0))


def _core_spec(d, core):
    return pl.BlockSpec((1, _RB, d), lambda i, c=core: (c, i, 0))


def _full_spec(shape):
    return pl.BlockSpec(shape, lambda i: tuple(0 for _ in shape))


def _gru_mats(w_ih, w_hh, b_ih, b_hh, h):
    z = jnp.zeros((h, h), jnp.float32)
    top = jnp.concatenate([w_ih.T, z], axis=1)
    wht = w_hh.T
    bot = jnp.concatenate([wht[:, :2 * h], z, wht[:, 2 * h:]], axis=1)
    wall = jnp.concatenate([top, bot], axis=0)
    ball = jnp.concatenate([b_ih[:2 * h] + b_hh[:2 * h], b_ih[2 * h:],
                            b_hh[2 * h:]]).reshape(1, 4 * h)
    return wall, ball


def kernel(x, edge_index, W1, b1, W2, b2, Wc1, bc1, gw_ih1, gw_hh1, gb_ih1,
           gb_hh1, Wc2, bc2, gw_ih2, gw_hh2, gb_ih2, gb_hh2, Wp, bp,
           prev1, prev2):
    pad = _EPAD - _E
    prow = ((jnp.arange(pad, dtype=jnp.int32) * 2503) % _N).reshape(1, -1, _BLK)
    pcol = (_N + (jnp.arange(pad, dtype=jnp.int32) % (_NP - _N))).reshape(1, -1, _BLK)
    ei = jnp.concatenate([edge_index.reshape(2, -1, _BLK),
                          jnp.concatenate([prow, pcol], axis=0)], axis=1)

    deg = _deg_call(ei)

    h1 = 64
    h2 = 32
    wall1, ball1 = _gru_mats(gw_ih1, gw_hh1, gb_ih1, gb_hh1, h1)
    wall2, ball2 = _gru_mats(gw_ih2, gw_hh2, gb_ih2, gb_hh2, h2)

    pk1 = pl.BlockSpec((_RB * h1 // 128, 128), lambda i: (i, 0))
    y1p = pl.pallas_call(
        _tc1_body,
        grid=(_GRID,),
        in_specs=[
            _row_spec(128), _full_spec((128, 256)), _full_spec((1, 256)),
            _full_spec((256, 128)), _full_spec((1, 128)),
            _full_spec((128, h1)), _core_spec(16, 0), _core_spec(16, 1),
        ],
        out_specs=pk1,
        out_shape=jax.ShapeDtypeStruct((_N * h1 // 128, 128), jnp.float32),
    )(x, W1, b1.reshape(1, 256), W2, b2.reshape(1, 128), Wc1, deg, deg)

    acc1 = _agg_call(ei, y1p.reshape(_N, h1), h1)

    pk2 = pl.BlockSpec((_RB * h2 // 128, 128), lambda i: (i, 0))
    emb1, y2p = pl.pallas_call(
        _tc2_body,
        grid=(_GRID,),
        in_specs=[
            _core_spec(128, 0), _core_spec(128, 1), pk1,
            _core_spec(16, 0), _core_spec(16, 1),
            pl.BlockSpec((h1, _RB), lambda i: (0, i)),
            _full_spec((1, h1)), _full_spec((2 * h1, 4 * h1)),
            _full_spec((1, 4 * h1)), _full_spec((h1, h2)),
        ],
        out_specs=[pl.BlockSpec((h1, _RB), lambda i: (0, i)), pk2],
        out_shape=[jax.ShapeDtypeStruct((h1, _N), jnp.float32),
                   jax.ShapeDtypeStruct((_N * h2 // 128, 128), jnp.float32)],
    )(acc1, acc1, y1p, deg, deg, prev1.T, bc1.reshape(1, h1),
      wall1, ball1, Wc2)

    acc2 = _agg_call(ei, y2p.reshape(_N, h2), h2)

    emb2, outv = pl.pallas_call(
        _tc3_body,
        grid=(_GRID,),
        in_specs=[
            _core_spec(128, 0), _core_spec(128, 1), pk2,
            _core_spec(16, 0), _core_spec(16, 1),
            pl.BlockSpec((h2, _RB), lambda i: (0, i)),
            _full_spec((1, h2)), _full_spec((2 * h2, 4 * h2)),
            _full_spec((1, 4 * h2)), _full_spec((h2, 2)), _full_spec((1, 2)),
        ],
        out_specs=[pl.BlockSpec((h2, _RB), lambda i: (0, i)),
                   pl.BlockSpec((_RB,), lambda i: (i,))],
        out_shape=[jax.ShapeDtypeStruct((h2, _N), jnp.float32),
                   jax.ShapeDtypeStruct((_N,), jnp.float32)],
    )(acc2, acc2, y2p, deg, deg, prev2.T, bc2.reshape(1, h2),
      wall2, ball2, Wp, bp.reshape(1, 2))

    return (outv, emb1.T, emb2.T)


# speedup vs baseline: 1.0012x; 1.0012x over previous
import functools

import numpy as np

import jax
import jax.numpy as jnp
from jax import lax
from jax.experimental import pallas as pl
from jax.experimental.pallas import tpu as pltpu
from jax.experimental.pallas import tpu_sc as plsc

_N = 10000
_E = 320000
_NP = 10240
_NW = 32
_BLK = 128
_S = 80
_EPAD = _NW * _S * _BLK
_RPT = _NP // 16
_RB = 1024
_GRID = 10

def _mesh():
    return plsc.VectorSubcoreMesh(core_axis_name="c", subcore_axis_name="s",
                                  num_cores=2, num_subcores=16)


def _leaky(v):
    return jnp.where(v > 0, v, 0.01 * v)


_W = 8


def _deg_body(ei_hbm, out_hbm, buf, cidx, acc, sems):
    c = lax.axis_index("c")
    t = lax.axis_index("s")
    w = c * 16 + t

    def fill(val):
        def body(r, _):
            buf[r, :] = jnp.full((16,), val, jnp.float32)
            return 0
        lax.fori_loop(0, _BLK, body, 0)

    fill(0.0)
    for k in range(_RPT // _BLK):
        pltpu.sync_copy(buf, acc.at[pl.ds(t * _RPT + k * _BLK, _BLK)])
    fill(1.0)
    pltpu.sync_copy(ei_hbm.at[1, pl.ds(w * _S, _S)], cidx)
    plsc.subcore_barrier()

    def scat(step, b):
        return pltpu.make_async_copy(buf, acc.at[cidx.at[step]], sems.at[b])

    for b in range(_W):
        scat(b, b).start(add=True)

    def group(i, _):
        for b in range(_W):
            scat(_W * i + b, b).wait()
            scat(_W * (i + 1) + b, b).start(add=True)
        return 0

    lax.fori_loop(0, _S // _W - 1, group, 0)
    for b in range(_W):
        scat(_S - _W + b, b).wait()
    plsc.subcore_barrier()
    pltpu.sync_copy(acc.at[pl.ds(t * _RPT, _RPT)],
                    out_hbm.at[c, pl.ds(t * _RPT, _RPT)])


def _deg_call(ei):
    fn = pl.kernel(
        _deg_body,
        out_type=jax.ShapeDtypeStruct((2, _NP, 16), jnp.float32),
        mesh=_mesh(),
        scratch_types=[
            pltpu.VMEM((_BLK, 16), jnp.float32),
            pltpu.VMEM((_S, _BLK), jnp.int32),
            pltpu.VMEM_SHARED((_NP, 16), jnp.float32),
            pltpu.SemaphoreType.DMA((_W,)),
        ],
        compiler_params=pltpu.CompilerParams(use_tc_tiling_on_sc=False),
    )
    return fn(ei)


def _agg_body(ei_hbm, y_hbm, out_hbm, ridx, cidx, *bufs_and_sems, d):
    gb = bufs_and_sems[:_W]
    acc = bufs_and_sems[_W]
    gsem = bufs_and_sems[_W + 1]
    ssem = bufs_and_sems[_W + 2]
    c = lax.axis_index("c")
    t = lax.axis_index("s")
    w = c * 16 + t

    def zrow(r, _):
        for j in range(d // 16):
            gb[0][r, pl.ds(j * 16, 16)] = jnp.zeros((16,), jnp.float32)
        return 0

    lax.fori_loop(0, _BLK, zrow, 0)
    for k in range(_RPT // _BLK):
        pltpu.sync_copy(gb[0], acc.at[pl.ds(t * _RPT + k * _BLK, _BLK)])
    pltpu.sync_copy(ei_hbm.at[0, pl.ds(w * _S, _S)], ridx)
    pltpu.sync_copy(ei_hbm.at[1, pl.ds(w * _S, _S)], cidx)
    plsc.subcore_barrier()

    def gath(step, b):
        return pltpu.make_async_copy(y_hbm.at[ridx.at[step]], gb[b],
                                     gsem.at[b])

    def scat(step, b):
        return pltpu.make_async_copy(gb[b], acc.at[cidx.at[step]],
                                     ssem.at[b])

    for b in range(_W):
        gath(b, b).start()

    def group(i, _):
        t0 = _W * i
        for b in range(_W):
            gath(t0 + b, b).wait()
            scat(t0 + b, b).start(add=True)
        for b in range(_W):
            scat(t0 + b, b).wait()
            gath(t0 + _W + b, b).start()
        return 0

    lax.fori_loop(0, _S // _W - 1, group, 0)
    t0 = _S - _W
    for b in range(_W):
        gath(t0 + b, b).wait()
        scat(t0 + b, b).start(add=True)
    for b in range(_W):
        scat(t0 + b, b).wait()
    plsc.subcore_barrier()
    pltpu.sync_copy(acc.at[pl.ds(t * _RPT, _RPT)],
                    out_hbm.at[c, pl.ds(t * _RPT, _RPT), pl.ds(0, d)])


def _agg_call(ei, y, d):
    fn = pl.kernel(
        functools.partial(_agg_body, d=d),
        out_type=jax.ShapeDtypeStruct((2, _NP, 128), jnp.float32),
        mesh=_mesh(),
        scratch_types=[
            pltpu.VMEM((_S, _BLK), jnp.int32),
            pltpu.VMEM((_S, _BLK), jnp.int32),
        ] + [pltpu.VMEM((_BLK, d), jnp.float32)] * _W + [
            pltpu.VMEM_SHARED((_NP, d), jnp.float32),
            pltpu.SemaphoreType.DMA((_W,)),
            pltpu.SemaphoreType.DMA((_W,)),
        ],
        compiler_params=pltpu.CompilerParams(use_tc_tiling_on_sc=False),
    )
    return fn(ei, y)


def _dis(dg0, dg1):
    return lax.rsqrt(dg0[0, :, 0:1] + dg1[0, :, 0:1] + 1.0)


def _pack(v, d):
    g = 128 // d
    r = v.reshape(v.shape[0] // g, g, d)
    return jnp.concatenate([r[:, k, :] for k in range(g)], axis=1)


def _unpack(p, d):
    g = 128 // d
    s = jnp.stack([p[:, k * d:(k + 1) * d] for k in range(g)], axis=1)
    return s.reshape(p.shape[0] * g, d)


def _tc1_body(x, w1, b1, w2, b2, wc1, dg0, dg1, y1):
    dis = _dis(dg0[:], dg1[:])
    h = _leaky(x[:] @ w1[:] + b1[:])
    h = _leaky(h @ w2[:] + b2[:])
    y1[:] = _pack((h @ wc1[:]) * dis, 64)


def _gru_block(g, p, wall, ball, h):
    m = jnp.concatenate([g, p], axis=1)
    gg = m @ wall + ball
    r = jax.nn.sigmoid(gg[:, :h])
    z = jax.nn.sigmoid(gg[:, h:2 * h])
    nn = jnp.tanh(gg[:, 2 * h:3 * h] + r * gg[:, 3 * h:])
    return (1.0 - z) * nn + z * p


def _tc2_body(a0, a1, y1, dg0, dg1, prev, bc, wall, ball, wc2, emb1, y2):
    dis = _dis(dg0[:], dg1[:])
    g = _leaky(dis * (a0[0][:, :64] + a1[0][:, :64] + _unpack(y1[:], 64))
               + bc[:])
    p = prev[:].T
    e = _gru_block(g, p, wall[:], ball[:], 64)
    emb1[:] = e.T
    y2[:] = _pack((e @ wc2[:]) * dis, 32)


def _tc3_body(a0, a1, y2, dg0, dg1, prev, bc, wall, ball, wp, bp, emb2, outv):
    dis = _dis(dg0[:], dg1[:])
    g = _leaky(dis * (a0[0][:, :32] + a1[0][:, :32] + _unpack(y2[:], 32))
               + bc[:])
    p = prev[:].T
    e = _gru_block(g, p, wall[:], ball[:], 32)
    emb2[:] = e.T
    outv[:] = jnp.sum(e @ wp[:] + bp[:], axis=1)


def _row_spec(d):
    return pl.BlockSpec((_RB, d), lambda i: (i, 0))


def _core_spec(d, core):
    return pl.BlockSpec((1, _RB, d), lambda i, c=core: (c, i, 0))


def _full_spec(shape):
    return pl.BlockSpec(shape, lambda i: tuple(0 for _ in shape))


def _gru_mats(w_ih, w_hh, b_ih, b_hh, h):
    z = jnp.zeros((h, h), jnp.float32)
    top = jnp.concatenate([w_ih.T, z], axis=1)
    wht = w_hh.T
    bot = jnp.concatenate([wht[:, :2 * h], z, wht[:, 2 * h:]], axis=1)
    wall = jnp.concatenate([top, bot], axis=0)
    ball = jnp.concatenate([b_ih[:2 * h] + b_hh[:2 * h], b_ih[2 * h:],
                            b_hh[2 * h:]]).reshape(1, 4 * h)
    return wall, ball


def kernel(x, edge_index, W1, b1, W2, b2, Wc1, bc1, gw_ih1, gw_hh1, gb_ih1,
           gb_hh1, Wc2, bc2, gw_ih2, gw_hh2, gb_ih2, gb_hh2, Wp, bp,
           prev1, prev2):
    pad = _EPAD - _E
    prow = (np.arange(pad, dtype=np.int32) * 2503) % _N
    pcol = _N + (np.arange(pad, dtype=np.int32) % (_NP - _N))
    epad = jnp.asarray(np.stack([prow, pcol]).reshape(2, -1, _BLK))
    ei = jnp.concatenate([edge_index.reshape(2, -1, _BLK), epad], axis=1)

    deg = _deg_call(ei)

    h1 = 64
    h2 = 32
    wall1, ball1 = _gru_mats(gw_ih1, gw_hh1, gb_ih1, gb_hh1, h1)
    wall2, ball2 = _gru_mats(gw_ih2, gw_hh2, gb_ih2, gb_hh2, h2)

    pk1 = pl.BlockSpec((_RB * h1 // 128, 128), lambda i: (i, 0))
    y1p = pl.pallas_call(
        _tc1_body,
        grid=(_GRID,),
        in_specs=[
            _row_spec(128), _full_spec((128, 256)), _full_spec((1, 256)),
            _full_spec((256, 128)), _full_spec((1, 128)),
            _full_spec((128, h1)), _core_spec(16, 0), _core_spec(16, 1),
        ],
        out_specs=pk1,
        out_shape=jax.ShapeDtypeStruct((_N * h1 // 128, 128), jnp.float32),
    )(x, W1, b1.reshape(1, 256), W2, b2.reshape(1, 128), Wc1, deg, deg)

    acc1 = _agg_call(ei, y1p.reshape(_N, h1), h1)

    pk2 = pl.BlockSpec((_RB * h2 // 128, 128), lambda i: (i, 0))
    emb1, y2p = pl.pallas_call(
        _tc2_body,
        grid=(_GRID,),
        in_specs=[
            _core_spec(128, 0), _core_spec(128, 1), pk1,
            _core_spec(16, 0), _core_spec(16, 1),
            pl.BlockSpec((h1, _RB), lambda i: (0, i)),
            _full_spec((1, h1)), _full_spec((2 * h1, 4 * h1)),
            _full_spec((1, 4 * h1)), _full_spec((h1, h2)),
        ],
        out_specs=[pl.BlockSpec((h1, _RB), lambda i: (0, i)), pk2],
        out_shape=[jax.ShapeDtypeStruct((h1, _N), jnp.float32),
                   jax.ShapeDtypeStruct((_N * h2 // 128, 128), jnp.float32)],
    )(acc1, acc1, y1p, deg, deg, prev1.T, bc1.reshape(1, h1),
      wall1, ball1, Wc2)

    acc2 = _agg_call(ei, y2p.reshape(_N, h2), h2)

    emb2, outv = pl.pallas_call(
        _tc3_body,
        grid=(_GRID,),
        in_specs=[
            _core_spec(128, 0), _core_spec(128, 1), pk2,
            _core_spec(16, 0), _core_spec(16, 1),
            pl.BlockSpec((h2, _RB), lambda i: (0, i)),
            _full_spec((1, h2)), _full_spec((2 * h2, 4 * h2)),
            _full_spec((1, 4 * h2)), _full_spec((h2, 2)), _full_spec((1, 2)),
        ],
        out_specs=[pl.BlockSpec((h2, _RB), lambda i: (0, i)),
                   pl.BlockSpec((_RB,), lambda i: (i,))],
        out_shape=[jax.ShapeDtypeStruct((h2, _N), jnp.float32),
                   jax.ShapeDtypeStruct((_N,), jnp.float32)],
    )(acc2, acc2, y2p, deg, deg, prev2.T, bc2.reshape(1, h2),
      wall2, ball2, Wp, bp.reshape(1, 2))

    return (outv, emb1.T, emb2.T)
